# Initial kernel scaffold; baseline (speedup 1.0000x reference)
import jax
import jax.numpy as jnp
from jax.experimental import pallas as pl


def kernel(t, x, A):
    raise NotImplementedError("write your pallas kernel here")


# speedup vs baseline: 5.6502x; 5.6502x over previous
import jax
import jax.numpy as jnp
from jax.experimental import pallas as pl

_N = 4096
_BM = 512


def _sis_kernel(a_ref, x_ref, xb_ref, o_ref):
    y = jnp.dot(a_ref[...], x_ref[...], preferred_element_type=jnp.float32)
    xb = xb_ref[...]
    o_ref[...] = (1.0 - xb) * y - xb


def kernel(t, x, A):
    grid = (_N // _BM,)
    out = pl.pallas_call(
        _sis_kernel,
        grid=grid,
        in_specs=[
            pl.BlockSpec((_BM, _N), lambda i: (i, 0)),
            pl.BlockSpec((_N, 1), lambda i: (0, 0)),
            pl.BlockSpec((_BM, 1), lambda i: (i, 0)),
        ],
        out_specs=pl.BlockSpec((_BM, 1), lambda i: (i, 0)),
        out_shape=jax.ShapeDtypeStruct((_N, 1), jnp.float32),
    )(A, x, x)
    return out
